# baseline (speedup 1.0000x reference)
import functools

import jax
import jax.numpy as jnp
from jax import lax
from jax.experimental import pallas as pl
from jax.experimental.pallas import tpu as pltpu
from jax.experimental.pallas import tpu_sc as plsc

B, T, N = 8, 12, 5000
NPAD = 5120
OUTW = 120
FT = OUTW // 8
NTILES = NPAD // 128
CN = 256
NCH_T = NPAD // CN
NCHUNKS = T * NCH_T
NW = 32
KMAX = -(-NCHUNKS // NW)
ROWB = NTILES * 8 * 128
OUT_DMA_BYTES = FT * 2048 * 4


def _ftbase(f):
    return f // 8, (f % 8) * 128


def _sc_body(x_hbm, wb_hbm, tab0_hbm, tab1_hbm, wtab_hbm, htab_hbm,
             node_hbm, adp_hbm, out_hbm,
             tab0_v, tab1_v, wtab_v, htab_v, wb_v,
             node_c, adp_c, x_c, buf0, buf1, sem0, sem1):
    wid = lax.axis_index("s") * 2 + lax.axis_index("c")

    pltpu.sync_copy(tab0_hbm, tab0_v)
    pltpu.sync_copy(tab1_hbm, tab1_v)
    pltpu.sync_copy(wtab_hbm, wtab_v.at[pl.ds(0, 8)])
    pltpu.sync_copy(htab_hbm, htab_v.at[pl.ds(0, 8)])
    pltpu.sync_copy(wb_hbm, wb_v)
    w_lo = wb_v[pl.ds(0, 16)]
    w_hi = wb_v[pl.ds(8, 16)]
    b_lo = wb_v[pl.ds(24, 16)]
    b_hi = wb_v[pl.ds(8 + 24, 16)]
    wvals = [w_lo[c] for c in range(16)] + [w_hi[c] for c in range(8, 16)]
    bvals = [b_lo[c] for c in range(16)] + [b_hi[c] for c in range(8, 16)]
    wt16 = wtab_v[pl.ds(0, 16)]
    ht16 = htab_v[pl.ds(0, 16)]
    wtvals = [(wt16[c], wt16[4 + c]) for c in range(4)]
    htvals = [(ht16[c], ht16[4 + c]) for c in range(4)]

    def _wait(buf, sem):
        pltpu.make_async_copy(buf, out_hbm.at[pl.ds(0, FT),
                                              pl.ds(0, 2048)], sem).wait()

    def chunk_body(k, _):
        ci = wid + k * NW

        @pl.when(ci < NCHUNKS)
        def _():
            t = ci // NCH_T
            m = ci % NCH_T
            n0 = m * CN

            @pl.when(k > 0)
            def _():
                _wait(buf0, sem0)
                _wait(buf1, sem1)

            pltpu.sync_copy(node_hbm.at[:, pl.ds(n0, CN)], node_c)
            pltpu.sync_copy(adp_hbm.at[t, :, pl.ds(n0, CN)], adp_c)
            pltpu.sync_copy(x_hbm.at[t, :, :, pl.ds(n0, CN)], x_c)

            def static_body(q, _):
                off = (q // 8) * 1024 + (q % 8) * 16
                src = pl.ds(q * 16, 16)
                for c in range(16):
                    r, cb = _ftbase(80 + c)
                    v = node_c[c, src]
                    buf0[r, pl.ds(cb + off, 16)] = v
                    buf1[r, pl.ds(cb + off, 16)] = v
                for c in range(24):
                    r, cb = _ftbase(96 + c)
                    v = adp_c[c, src]
                    buf0[r, pl.ds(cb + off, 16)] = v
                    buf1[r, pl.ds(cb + off, 16)] = v
                return 0

            lax.fori_loop(0, 16, static_body, 0)

            def do_batch(b, buf, sem):
                def dyn_one(q, buf, b):
                    off = (q // 8) * 1024 + (q % 8) * 16
                    src = pl.ds(q * 16, 16)
                    x0 = x_c[0, b, src]
                    x1 = x_c[1, b, src]
                    x2 = x_c[2, b, src]
                    x3 = x_c[3, b, src]
                    x4 = x_c[4, b, src]
                    i0 = (x1 * 288.0).astype(jnp.int32) * 25
                    i1 = (x2 * 7.0).astype(jnp.int32) * 25
                    wk0 = x3 < 1.0
                    hd0 = x4 < 1.0
                    for c in range(24):
                        r, cb = _ftbase(c)
                        buf[r, pl.ds(cb + off, 16)] = x0 * wvals[c] + bvals[c]
                    for c in range(24):
                        r, cb = _ftbase(24 + c)
                        buf[r, pl.ds(cb + off, 16)] = plsc.load_gather(
                            tab0_v, (i0 + c,))
                    for c in range(24):
                        r, cb = _ftbase(48 + c)
                        buf[r, pl.ds(cb + off, 16)] = plsc.load_gather(
                            tab1_v, (i1 + c,))
                    for c in range(4):
                        r, cb = _ftbase(72 + c)
                        buf[r, pl.ds(cb + off, 16)] = jnp.where(
                            wk0, wtvals[c][0], wtvals[c][1])
                    for c in range(4):
                        r, cb = _ftbase(76 + c)
                        buf[r, pl.ds(cb + off, 16)] = jnp.where(
                            hd0, htvals[c][0], htvals[c][1])

                def dyn_body(q4, _):
                    for u in range(4):
                        dyn_one(q4 * 4 + u, buf, b)
                    return 0

                lax.fori_loop(0, 4, dyn_body, 0)
                row0 = (b * T + t) * FT
                pltpu.async_copy(
                    buf, out_hbm.at[pl.ds(row0, FT), pl.ds(m * 2048, 2048)],
                    sem)

            do_batch(0, buf0, sem0)
            do_batch(1, buf1, sem1)

        return 0

    lax.fori_loop(0, KMAX, chunk_body, 0)
    _wait(buf0, sem0)
    _wait(buf1, sem1)


def _encode(x, W_in, b_in, period_tab0, period_tab1, weekend_tab,
            holiday_tab, node_emb, adp_emb):
    pad = NPAD - N
    xt = jnp.pad(x.transpose(1, 3, 0, 2), ((0, 0), (0, 0), (0, 0), (0, pad)))
    adpt = jnp.pad(adp_emb.transpose(0, 2, 1), ((0, 0), (0, 0), (0, pad)))
    nodet = jnp.pad(node_emb.T, ((0, 0), (0, pad)))
    wb = jnp.concatenate([W_in.reshape(24), b_in])
    mesh = plsc.VectorSubcoreMesh(core_axis_name="c", subcore_axis_name="s")
    run = pl.kernel(
        _sc_body,
        out_type=jax.ShapeDtypeStruct((2 * T * FT, ROWB), jnp.float32),
        mesh=mesh,
        compiler_params=pltpu.CompilerParams(needs_layout_passes=False,
                                             use_tc_tiling_on_sc=False),
        scratch_types=[
            pltpu.VMEM((288 * 25,), jnp.float32),
            pltpu.VMEM((7 * 25,), jnp.float32),
            pltpu.VMEM((16,), jnp.float32),
            pltpu.VMEM((16,), jnp.float32),
            pltpu.VMEM((48,), jnp.float32),
            pltpu.VMEM((16, CN), jnp.float32),
            pltpu.VMEM((24, CN), jnp.float32),
            pltpu.VMEM((5, 2, CN), jnp.float32),
            pltpu.VMEM((FT, 2048), jnp.float32),
            pltpu.VMEM((FT, 2048), jnp.float32),
            pltpu.SemaphoreType.DMA,
            pltpu.SemaphoreType.DMA,
        ],
    )
    tab0p = jnp.pad(period_tab0, ((0, 0), (0, 1))).reshape(288 * 25)
    tab1p = jnp.pad(period_tab1, ((0, 0), (0, 1))).reshape(7 * 25)
    pieces = []
    for i in range(4):
        out = run(xt[:, :, 2 * i:2 * i + 2, :], wb, tab0p, tab1p,
                  weekend_tab.reshape(8), holiday_tab.reshape(8),
                  nodet, adpt)
        o6 = out.reshape(2, T, FT, NTILES, 8, 128)
        o6 = o6.transpose(0, 1, 2, 4, 3, 5)
        o4 = o6.reshape(2, T, OUTW, NPAD)[:, :, :, :N]
        pieces.append(o4.transpose(0, 1, 3, 2))
    return jnp.concatenate(pieces, axis=0)


_jitted = jax.jit(_encode)


def kernel(x, W_in, b_in, period_tab0, period_tab1, weekend_tab,
           holiday_tab, node_emb, adp_emb):
    return _jitted(x, W_in, b_in, period_tab0, period_tab1, weekend_tab,
                   holiday_tab, node_emb, adp_emb)


# speedup vs baseline: 1.5872x; 1.5872x over previous
import functools

import jax
import jax.numpy as jnp
from jax import lax
from jax.experimental import pallas as pl
from jax.experimental.pallas import tpu as pltpu
from jax.experimental.pallas import tpu_sc as plsc

B, T, N = 8, 12, 5000
NPAD = 5120
OUTW = 120
FT = OUTW // 8
NTILES = NPAD // 128
CN = 256
NCH_T = NPAD // CN
NCHUNKS = T * NCH_T
NW = 32
KMAX = -(-NCHUNKS // NW)
ROWB = NTILES * 8 * 128
OUT_DMA_BYTES = FT * 2048 * 4


def _ftbase(f):
    return f // 8, (f % 8) * 128


def _sc_body(x_hbm, wb_hbm, tab0_hbm, tab1_hbm, wtab_hbm, htab_hbm,
             node_hbm, adp_hbm, out_hbm,
             tab0_v, tab1_v, wtab_v, htab_v, wb_v,
             node_c, adp_c, x_c, buf0, buf1, sem0, sem1):
    wid = lax.axis_index("s") * 2 + lax.axis_index("c")

    pltpu.sync_copy(tab0_hbm, tab0_v)
    pltpu.sync_copy(tab1_hbm, tab1_v)
    pltpu.sync_copy(wtab_hbm, wtab_v.at[pl.ds(0, 8)])
    pltpu.sync_copy(htab_hbm, htab_v.at[pl.ds(0, 8)])
    pltpu.sync_copy(wb_hbm, wb_v)
    w_lo = wb_v[pl.ds(0, 16)]
    w_hi = wb_v[pl.ds(8, 16)]
    b_lo = wb_v[pl.ds(24, 16)]
    b_hi = wb_v[pl.ds(8 + 24, 16)]
    wvals = [w_lo[c] for c in range(16)] + [w_hi[c] for c in range(8, 16)]
    bvals = [b_lo[c] for c in range(16)] + [b_hi[c] for c in range(8, 16)]
    wt16 = wtab_v[pl.ds(0, 16)]
    ht16 = htab_v[pl.ds(0, 16)]
    wtvals = [(wt16[c], wt16[4 + c]) for c in range(4)]
    htvals = [(ht16[c], ht16[4 + c]) for c in range(4)]

    def _wait(buf, sem):
        pltpu.make_async_copy(buf, out_hbm.at[pl.ds(0, FT),
                                              pl.ds(0, 2048)], sem).wait()

    def chunk_body(k, _):
        ci = wid + k * NW

        @pl.when(ci < NCHUNKS)
        def _():
            t = ci // NCH_T
            m = ci % NCH_T
            n0 = m * CN

            @pl.when(k > 0)
            def _():
                _wait(buf0, sem0)
                _wait(buf1, sem1)

            pltpu.sync_copy(node_hbm.at[:, pl.ds(n0, CN)], node_c)
            pltpu.sync_copy(adp_hbm.at[t, :, pl.ds(n0, CN)], adp_c)
            pltpu.sync_copy(x_hbm.at[t, :, :, pl.ds(n0, CN)], x_c)

            def static_body(q, _):
                off = (q // 8) * 1024 + (q % 8) * 16
                src = pl.ds(q * 16, 16)
                for c in range(16):
                    r, cb = _ftbase(80 + c)
                    v = node_c[c, src]
                    buf0[r, pl.ds(cb + off, 16)] = v
                    buf1[r, pl.ds(cb + off, 16)] = v
                for c in range(24):
                    r, cb = _ftbase(96 + c)
                    v = adp_c[c, src]
                    buf0[r, pl.ds(cb + off, 16)] = v
                    buf1[r, pl.ds(cb + off, 16)] = v
                return 0

            lax.fori_loop(0, 16, static_body, 0)

            def do_batch(b, buf, sem):
                def dyn_one(q, buf, b):
                    off = (q // 8) * 1024 + (q % 8) * 16
                    src = pl.ds(q * 16, 16)
                    x0 = x_c[0, b, src]
                    x1 = x_c[1, b, src]
                    x2 = x_c[2, b, src]
                    x3 = x_c[3, b, src]
                    x4 = x_c[4, b, src]
                    i0 = (x1 * 288.0).astype(jnp.int32) * 25
                    i1 = (x2 * 7.0).astype(jnp.int32) * 25
                    wk0 = x3 < 1.0
                    hd0 = x4 < 1.0
                    for c in range(24):
                        r, cb = _ftbase(c)
                        buf[r, pl.ds(cb + off, 16)] = x0 * wvals[c] + bvals[c]
                    for c in range(24):
                        r, cb = _ftbase(24 + c)
                        buf[r, pl.ds(cb + off, 16)] = plsc.load_gather(
                            tab0_v, (i0 + c,))
                    for c in range(24):
                        r, cb = _ftbase(48 + c)
                        buf[r, pl.ds(cb + off, 16)] = plsc.load_gather(
                            tab1_v, (i1 + c,))
                    for c in range(4):
                        r, cb = _ftbase(72 + c)
                        buf[r, pl.ds(cb + off, 16)] = jnp.where(
                            wk0, wtvals[c][0], wtvals[c][1])
                    for c in range(4):
                        r, cb = _ftbase(76 + c)
                        buf[r, pl.ds(cb + off, 16)] = jnp.where(
                            hd0, htvals[c][0], htvals[c][1])

                def dyn_body(q4, _):
                    for u in range(4):
                        dyn_one(q4 * 4 + u, buf, b)
                    return 0

                lax.fori_loop(0, 4, dyn_body, 0)
                row0 = (b * T + t) * FT
                pltpu.async_copy(
                    buf, out_hbm.at[pl.ds(row0, FT), pl.ds(m * 2048, 2048)],
                    sem)

            def b_body(j, _):
                @pl.when(j > 0)
                def _():
                    _wait(buf0, sem0)
                do_batch(2 * j, buf0, sem0)

                @pl.when(j > 0)
                def _():
                    _wait(buf1, sem1)
                do_batch(2 * j + 1, buf1, sem1)
                return 0

            lax.fori_loop(0, 4, b_body, 0)

        return 0

    lax.fori_loop(0, KMAX, chunk_body, 0)
    _wait(buf0, sem0)
    _wait(buf1, sem1)


def _encode(x, W_in, b_in, period_tab0, period_tab1, weekend_tab,
            holiday_tab, node_emb, adp_emb):
    pad = NPAD - N
    xt = jnp.pad(x.transpose(1, 3, 0, 2), ((0, 0), (0, 0), (0, 0), (0, pad)))
    adpt = jnp.pad(adp_emb.transpose(0, 2, 1), ((0, 0), (0, 0), (0, pad)))
    nodet = jnp.pad(node_emb.T, ((0, 0), (0, pad)))
    wb = jnp.concatenate([W_in.reshape(24), b_in])
    mesh = plsc.VectorSubcoreMesh(core_axis_name="c", subcore_axis_name="s")
    run = pl.kernel(
        _sc_body,
        out_type=jax.ShapeDtypeStruct((B * T * FT, ROWB), jnp.float32),
        mesh=mesh,
        compiler_params=pltpu.CompilerParams(needs_layout_passes=False,
                                             use_tc_tiling_on_sc=False),
        scratch_types=[
            pltpu.VMEM((288 * 25,), jnp.float32),
            pltpu.VMEM((7 * 25,), jnp.float32),
            pltpu.VMEM((16,), jnp.float32),
            pltpu.VMEM((16,), jnp.float32),
            pltpu.VMEM((48,), jnp.float32),
            pltpu.VMEM((16, CN), jnp.float32),
            pltpu.VMEM((24, CN), jnp.float32),
            pltpu.VMEM((5, B, CN), jnp.float32),
            pltpu.VMEM((FT, 2048), jnp.float32),
            pltpu.VMEM((FT, 2048), jnp.float32),
            pltpu.SemaphoreType.DMA,
            pltpu.SemaphoreType.DMA,
        ],
    )
    tab0p = jnp.pad(period_tab0, ((0, 0), (0, 1))).reshape(288 * 25)
    tab1p = jnp.pad(period_tab1, ((0, 0), (0, 1))).reshape(7 * 25)
    out = run(xt, wb, tab0p, tab1p, weekend_tab.reshape(8),
              holiday_tab.reshape(8), nodet, adpt)
    o6 = out.reshape(B, T, FT, NTILES, 8, 128)
    o6 = o6.transpose(0, 1, 2, 4, 3, 5)
    o4 = o6.reshape(B, T, OUTW, NPAD)[:, :, :, :N]
    return o4.transpose(0, 1, 3, 2)


_jitted = jax.jit(_encode)


def kernel(x, W_in, b_in, period_tab0, period_tab1, weekend_tab,
           holiday_tab, node_emb, adp_emb):
    return _jitted(x, W_in, b_in, period_tab0, period_tab1, weekend_tab,
                   holiday_tab, node_emb, adp_emb)
